# baseline (speedup 1.0000x reference)
import functools

import jax
import jax.numpy as jnp
from jax import lax
from jax.experimental import pallas as pl
from jax.experimental.pallas import tpu as pltpu
from jax.experimental.pallas import tpu_sc as plsc

_L = 16
_NC = 2
_NS = 16
_NW = _NC * _NS
_OUT_CHUNK = 8192


def _sc_gather_body(mus_hbm, sig_hbm, z_hbm, muz_hbm, sigz_hbm,
                    z_v, row_v, out_v):
    D, N = mus_hbm.shape
    B = z_hbm.shape[0]
    rows_per_w = D // _NW

    wid = lax.axis_index("s") * _NC + lax.axis_index("c")

    pltpu.sync_copy(z_hbm, z_v)

    n_chunks = B // _OUT_CHUNK
    steps = _OUT_CHUNK // _L

    for src, dst in ((mus_hbm, muz_hbm), (sig_hbm, sigz_hbm)):
        for r in range(rows_per_w):
            d = wid * rows_per_w + r
            pltpu.sync_copy(src.at[d], row_v)
            for h in range(n_chunks):
                def gather_step(j, carry, h=h):
                    idx = z_v[pl.ds(h * _OUT_CHUNK + j * _L, _L)]
                    out_v[pl.ds(j * _L, _L)] = plsc.load_gather(row_v, [idx])
                    return carry
                lax.fori_loop(0, steps, gather_step, 0)
                pltpu.sync_copy(out_v, dst.at[d, pl.ds(h * _OUT_CHUNK, _OUT_CHUNK)])


def kernel(mus, sigmas, z):
    D, N = mus.shape
    B = z.shape[0]
    out = jax.ShapeDtypeStruct((D, B), jnp.float32)
    mesh = plsc.VectorSubcoreMesh(core_axis_name="c", subcore_axis_name="s")
    k = functools.partial(
        pl.kernel,
        out_type=(out, out),
        mesh=mesh,
        scratch_types=[
            pltpu.VMEM((B,), jnp.int32),
            pltpu.VMEM((N,), jnp.float32),
            pltpu.VMEM((_OUT_CHUNK,), jnp.float32),
        ],
        compiler_params=pltpu.CompilerParams(needs_layout_passes=False),
    )(_sc_gather_body)
    return k(mus, sigmas, z)


# speedup vs baseline: 1.3097x; 1.3097x over previous
import functools

import jax
import jax.numpy as jnp
from jax import lax
from jax.experimental import pallas as pl
from jax.experimental.pallas import tpu as pltpu
from jax.experimental.pallas import tpu_sc as plsc

_L = 16
_NC = 2
_NS = 16
_NW = _NC * _NS
_OUT_CHUNK = 4096


def _sc_gather_body(mus_hbm, sig_hbm, z_hbm, muz_hbm, sigz_hbm,
                    z_v, row_v, out_v, sem0, sem1):
    D, N = mus_hbm.shape
    B = z_hbm.shape[0]
    rows_per_w = D // _NW

    wid = lax.axis_index("s") * _NC + lax.axis_index("c")

    pltpu.sync_copy(z_hbm, z_v)

    n_chunks = B // _OUT_CHUNK
    sems = (sem0, sem1)
    pending = [None, None]

    for src, dst in ((mus_hbm, muz_hbm), (sig_hbm, sigz_hbm)):
        for r in range(rows_per_w):
            d = wid * rows_per_w + r
            pltpu.sync_copy(src.at[d], row_v)
            for h in range(n_chunks):
                b = h % 2
                if pending[b] is not None:
                    pending[b].wait()
                    pending[b] = None

                @plsc.parallel_loop(0, _OUT_CHUNK, step=_L, unroll=8)
                def gather_step(j, h=h, b=b):
                    idx = z_v[pl.ds(h * _OUT_CHUNK + j, _L)]
                    out_v[b, pl.ds(j, _L)] = plsc.load_gather(row_v, [idx])

                pending[b] = pltpu.async_copy(
                    out_v.at[b],
                    dst.at[d, pl.ds(h * _OUT_CHUNK, _OUT_CHUNK)],
                    sems[b],
                )
    for b in range(2):
        if pending[b] is not None:
            pending[b].wait()


def kernel(mus, sigmas, z):
    D, N = mus.shape
    B = z.shape[0]
    out = jax.ShapeDtypeStruct((D, B), jnp.float32)
    mesh = plsc.VectorSubcoreMesh(core_axis_name="c", subcore_axis_name="s")
    k = functools.partial(
        pl.kernel,
        out_type=(out, out),
        mesh=mesh,
        scratch_types=[
            pltpu.VMEM((B,), jnp.int32),
            pltpu.VMEM((N,), jnp.float32),
            pltpu.VMEM((2, _OUT_CHUNK), jnp.float32),
            pltpu.SemaphoreType.DMA,
            pltpu.SemaphoreType.DMA,
        ],
        compiler_params=pltpu.CompilerParams(needs_layout_passes=False),
    )(_sc_gather_body)
    return k(mus, sigmas, z)
